# baseline (speedup 1.0000x reference)
import functools

import jax
import jax.numpy as jnp
from jax import lax
from jax.experimental import pallas as pl
from jax.experimental.pallas import tpu as pltpu
from jax.experimental.pallas import tpu_sc as plsc

BATCH = 16384
EMB_K = 16
NUM_WORKERS = 32
ROWS_PER_WORKER = BATCH // NUM_WORKERS
GROUP = 16
NUM_GROUPS = ROWS_PER_WORKER // GROUP


def _gather_body(uidx_hbm, vidx_hbm, w_hbm, h_hbm, u_out, v_out,
                 uidx_v, vidx_v, sem_u, sem_v):
    wid = lax.axis_index("s") * 2 + lax.axis_index("c")
    base = wid * ROWS_PER_WORKER
    pltpu.sync_copy(uidx_hbm.at[pl.ds(base, ROWS_PER_WORKER)], uidx_v)
    pltpu.sync_copy(vidx_hbm.at[pl.ds(base, ROWS_PER_WORKER)], vidx_v)

    def fire(g):
        uvec = uidx_v[pl.ds(g * GROUP, GROUP)]
        vvec = vidx_v[pl.ds(g * GROUP, GROUP)]
        for j in range(GROUP):
            i = g * GROUP + j
            ru = uvec[j]
            rv = vvec[j]
            pltpu.make_async_copy(
                w_hbm.at[pl.ds(ru, 1)], u_out.at[pl.ds(base + i, 1)],
                sem_u).start()
            pltpu.make_async_copy(
                h_hbm.at[pl.ds(rv, 1)], v_out.at[pl.ds(base + i, 1)],
                sem_v).start()

    def drain(g):
        pltpu.make_async_copy(
            w_hbm.at[pl.ds(0, GROUP)],
            u_out.at[pl.ds(base + g * GROUP, GROUP)], sem_u).wait()
        pltpu.make_async_copy(
            h_hbm.at[pl.ds(0, GROUP)],
            v_out.at[pl.ds(base + g * GROUP, GROUP)], sem_v).wait()

    fire(0)

    def loop_body(g, _):
        fire_g = g + 1

        @pl.when(fire_g < NUM_GROUPS)
        def _():
            fire(fire_g)

        drain(g)
        return ()

    lax.fori_loop(0, NUM_GROUPS, loop_body, (), unroll=False)


_gather_call = functools.partial(
    pl.kernel,
    out_type=(
        jax.ShapeDtypeStruct((BATCH, EMB_K), jnp.float32),
        jax.ShapeDtypeStruct((BATCH, EMB_K), jnp.float32),
    ),
    mesh=plsc.VectorSubcoreMesh(core_axis_name="c", subcore_axis_name="s"),
    scratch_types=[
        pltpu.VMEM((ROWS_PER_WORKER,), jnp.int32),
        pltpu.VMEM((ROWS_PER_WORKER,), jnp.int32),
        pltpu.SemaphoreType.DMA,
        pltpu.SemaphoreType.DMA,
    ],
)(_gather_body)


def _mlp_body(u_ref, v_ref, w1_ref, b1_ref, w2_ref, o_ref):
    u = u_ref[...]
    v = v_ref[...]
    w1a = w1_ref[0:EMB_K, :]
    w1b = w1_ref[EMB_K:2 * EMB_K, :]
    h = jnp.dot(u, w1a, preferred_element_type=jnp.float32)
    h = h + jnp.dot(v, w1b, preferred_element_type=jnp.float32)
    h = jnp.maximum(h + b1_ref[...], 0.0)
    o_ref[...] = jnp.sum(h * w2_ref[...], axis=1, keepdims=True)


def _mlp_call(u, v, w1, b1_row, w2_row):
    return pl.pallas_call(
        _mlp_body,
        out_shape=jax.ShapeDtypeStruct((BATCH, 1), jnp.float32),
    )(u, v, w1, b1_row, w2_row)


def kernel(x, W, H, W1, b1, W2):
    uidx = x[:, 0].astype(jnp.int32)
    vidx = x[:, 1].astype(jnp.int32)
    u_rows, v_rows = _gather_call(uidx, vidx, W, H)
    return _mlp_call(u_rows, v_rows, W1, b1.reshape(1, EMB_K),
                     W2.reshape(1, EMB_K))


# speedup vs baseline: 1.8637x; 1.8637x over previous
import functools

import jax
import jax.numpy as jnp
from jax import lax
from jax.experimental import pallas as pl
from jax.experimental.pallas import tpu as pltpu
from jax.experimental.pallas import tpu_sc as plsc

BATCH = 16384
EMB_K = 16
NUM_WORKERS = 32
ROWS_PER_WORKER = BATCH // NUM_WORKERS
GROUP = 16
CHUNK = 256
NUM_PASSES = ROWS_PER_WORKER // CHUNK
GROUPS_PER_PASS = CHUNK // GROUP


def _gather_body(uidx_hbm, vidx_hbm, w_hbm, h_hbm, u_out, v_out,
                 uidx_v, vidx_v, u_v, v_v, sem_u, sem_v):
    wid = lax.axis_index("s") * 2 + lax.axis_index("c")
    base = wid * ROWS_PER_WORKER
    pltpu.sync_copy(uidx_hbm.at[pl.ds(base, ROWS_PER_WORKER)], uidx_v)
    pltpu.sync_copy(vidx_hbm.at[pl.ds(base, ROWS_PER_WORKER)], vidx_v)

    for p in range(NUM_PASSES):
        def fire(g, p=p):
            uvec = uidx_v[pl.ds(p * CHUNK + g * GROUP, GROUP)]
            vvec = vidx_v[pl.ds(p * CHUNK + g * GROUP, GROUP)]
            for j in range(GROUP):
                i = g * GROUP + j
                ru = uvec[j]
                rv = vvec[j]
                pltpu.make_async_copy(
                    w_hbm.at[pl.ds(ru, 1)], u_v.at[pl.ds(i, 1)],
                    sem_u).start()
                pltpu.make_async_copy(
                    h_hbm.at[pl.ds(rv, 1)], v_v.at[pl.ds(i, 1)],
                    sem_v).start()

        def drain(g):
            pltpu.make_async_copy(
                w_hbm.at[pl.ds(0, GROUP)],
                u_v.at[pl.ds(g * GROUP, GROUP)], sem_u).wait()
            pltpu.make_async_copy(
                h_hbm.at[pl.ds(0, GROUP)],
                v_v.at[pl.ds(g * GROUP, GROUP)], sem_v).wait()

        fire(0)

        def loop_body(g, _):
            fire_g = g + 1

            @pl.when(fire_g < GROUPS_PER_PASS)
            def _():
                fire(fire_g)

            drain(g)
            return ()

        lax.fori_loop(0, GROUPS_PER_PASS, loop_body, (), unroll=False)

        pltpu.sync_copy(u_v, u_out.at[pl.ds(base + p * CHUNK, CHUNK)])
        pltpu.sync_copy(v_v, v_out.at[pl.ds(base + p * CHUNK, CHUNK)])


_gather_call = functools.partial(
    pl.kernel,
    out_type=(
        jax.ShapeDtypeStruct((BATCH, EMB_K), jnp.float32),
        jax.ShapeDtypeStruct((BATCH, EMB_K), jnp.float32),
    ),
    mesh=plsc.VectorSubcoreMesh(core_axis_name="c", subcore_axis_name="s"),
    scratch_types=[
        pltpu.VMEM((ROWS_PER_WORKER,), jnp.int32),
        pltpu.VMEM((ROWS_PER_WORKER,), jnp.int32),
        pltpu.VMEM((CHUNK, EMB_K), jnp.float32),
        pltpu.VMEM((CHUNK, EMB_K), jnp.float32),
        pltpu.SemaphoreType.DMA,
        pltpu.SemaphoreType.DMA,
    ],
)(_gather_body)


def _mlp_body(u_ref, v_ref, w1_ref, b1_ref, w2_ref, o_ref):
    u = u_ref[...]
    v = v_ref[...]
    w1a = w1_ref[0:EMB_K, :]
    w1b = w1_ref[EMB_K:2 * EMB_K, :]
    h = jnp.dot(u, w1a, preferred_element_type=jnp.float32)
    h = h + jnp.dot(v, w1b, preferred_element_type=jnp.float32)
    h = jnp.maximum(h + b1_ref[...], 0.0)
    o_ref[...] = jnp.sum(h * w2_ref[...], axis=1, keepdims=True)


def _mlp_call(u, v, w1, b1_row, w2_row):
    return pl.pallas_call(
        _mlp_body,
        out_shape=jax.ShapeDtypeStruct((BATCH, 1), jnp.float32),
    )(u, v, w1, b1_row, w2_row)


def kernel(x, W, H, W1, b1, W2):
    uidx = x[:, 0].astype(jnp.int32)
    vidx = x[:, 1].astype(jnp.int32)
    u_rows, v_rows = _gather_call(uidx, vidx, W, H)
    return _mlp_call(u_rows, v_rows, W1, b1.reshape(1, EMB_K),
                     W2.reshape(1, EMB_K))


# speedup vs baseline: 1.8798x; 1.0086x over previous
import functools

import jax
import jax.numpy as jnp
from jax import lax
from jax.experimental import pallas as pl
from jax.experimental.pallas import tpu as pltpu
from jax.experimental.pallas import tpu_sc as plsc

BATCH = 16384
EMB_K = 16
NUM_WORKERS = 32
ROWS_PER_WORKER = BATCH // NUM_WORKERS
GROUP = 16
LAG = 4
CHUNK = 256
NUM_PASSES = ROWS_PER_WORKER // CHUNK
GROUPS_PER_PASS = CHUNK // GROUP


def _gather_body(uidx_hbm, vidx_hbm, w_hbm, h_hbm, u_out, v_out,
                 uidx_v, vidx_v, u_v, v_v, sem_u, sem_v):
    wid = lax.axis_index("s") * 2 + lax.axis_index("c")
    base = wid * ROWS_PER_WORKER
    pltpu.sync_copy(uidx_hbm.at[pl.ds(base, ROWS_PER_WORKER)], uidx_v)
    pltpu.sync_copy(vidx_hbm.at[pl.ds(base, ROWS_PER_WORKER)], vidx_v)

    for p in range(NUM_PASSES):
        def fire(g, p=p):
            uvec = uidx_v[pl.ds(p * CHUNK + g * GROUP, GROUP)]
            vvec = vidx_v[pl.ds(p * CHUNK + g * GROUP, GROUP)]
            for j in range(GROUP):
                i = g * GROUP + j
                ru = uvec[j]
                rv = vvec[j]
                pltpu.make_async_copy(
                    w_hbm.at[pl.ds(ru, 1)], u_v.at[pl.ds(i, 1)],
                    sem_u).start()
                pltpu.make_async_copy(
                    h_hbm.at[pl.ds(rv, 1)], v_v.at[pl.ds(i, 1)],
                    sem_v).start()

        def drain(g):
            pltpu.make_async_copy(
                w_hbm.at[pl.ds(0, GROUP)],
                u_v.at[pl.ds(g * GROUP, GROUP)], sem_u).wait()
            pltpu.make_async_copy(
                h_hbm.at[pl.ds(0, GROUP)],
                v_v.at[pl.ds(g * GROUP, GROUP)], sem_v).wait()

        for g0 in range(LAG):
            fire(g0)

        def loop_body(g, _):
            fire_g = g + LAG

            @pl.when(fire_g < GROUPS_PER_PASS)
            def _():
                fire(fire_g)

            drain(g)
            return ()

        lax.fori_loop(0, GROUPS_PER_PASS, loop_body, (), unroll=False)

        pltpu.sync_copy(u_v, u_out.at[pl.ds(base + p * CHUNK, CHUNK)])
        pltpu.sync_copy(v_v, v_out.at[pl.ds(base + p * CHUNK, CHUNK)])


_gather_call = functools.partial(
    pl.kernel,
    out_type=(
        jax.ShapeDtypeStruct((BATCH, EMB_K), jnp.float32),
        jax.ShapeDtypeStruct((BATCH, EMB_K), jnp.float32),
    ),
    mesh=plsc.VectorSubcoreMesh(core_axis_name="c", subcore_axis_name="s"),
    scratch_types=[
        pltpu.VMEM((ROWS_PER_WORKER,), jnp.int32),
        pltpu.VMEM((ROWS_PER_WORKER,), jnp.int32),
        pltpu.VMEM((CHUNK, EMB_K), jnp.float32),
        pltpu.VMEM((CHUNK, EMB_K), jnp.float32),
        pltpu.SemaphoreType.DMA,
        pltpu.SemaphoreType.DMA,
    ],
)(_gather_body)


def _mlp_body(u_ref, v_ref, w1_ref, b1_ref, w2_ref, o_ref):
    u = u_ref[...]
    v = v_ref[...]
    w1a = w1_ref[0:EMB_K, :]
    w1b = w1_ref[EMB_K:2 * EMB_K, :]
    h = jnp.dot(u, w1a, preferred_element_type=jnp.float32)
    h = h + jnp.dot(v, w1b, preferred_element_type=jnp.float32)
    h = jnp.maximum(h + b1_ref[...], 0.0)
    o_ref[...] = jnp.sum(h * w2_ref[...], axis=1, keepdims=True)


def _mlp_call(u, v, w1, b1_row, w2_row):
    return pl.pallas_call(
        _mlp_body,
        out_shape=jax.ShapeDtypeStruct((BATCH, 1), jnp.float32),
    )(u, v, w1, b1_row, w2_row)


def kernel(x, W, H, W1, b1, W2):
    uidx = x[:, 0].astype(jnp.int32)
    vidx = x[:, 1].astype(jnp.int32)
    u_rows, v_rows = _gather_call(uidx, vidx, W, H)
    return _mlp_call(u_rows, v_rows, W1, b1.reshape(1, EMB_K),
                     W2.reshape(1, EMB_K))
